# baseline (speedup 1.0000x reference)
from math import pi as PI, sqrt

import jax
import jax.numpy as jnp
from jax.experimental import pallas as pl

_RHO = 1.0
_U = 0.0
_T = 1.0
_PREFACTOR = _RHO / sqrt(2 * PI * _T)


def _gauss_body(v_ref, o_ref):
    v = v_ref[...]
    o_ref[...] = _PREFACTOR * jnp.exp(-((v - _U) ** 2) / (2 * _T))


def kernel(txv):
    n = txv.shape[0]
    s = jnp.sort(txv[:, 2])
    keep = jnp.concatenate([jnp.ones((1,), bool), s[1:] != s[:-1]])
    pos = jnp.cumsum(keep) - 1
    u = jnp.zeros((n,), jnp.float32).at[pos].set(s)
    u2 = u.reshape(n // 1024, 1024)
    out = pl.pallas_call(
        _gauss_body,
        out_shape=jax.ShapeDtypeStruct(u2.shape, jnp.float32),
        grid=(n // 1024 // 256,),
        in_specs=[pl.BlockSpec((256, 1024), lambda i: (i, 0))],
        out_specs=pl.BlockSpec((256, 1024), lambda i: (i, 0)),
    )(u2)
    return out.reshape(n)
